# baseline (speedup 1.0000x reference)
import functools

import jax
import jax.numpy as jnp
from jax.experimental import pallas as pl
from jax.experimental.pallas import tpu as pltpu

_ALPHA = 0.2


def _d2_body(inp_ref, tgt_ref, out_ref):
    d = inp_ref[...] - tgt_ref[...]
    out_ref[...] = d * d


def _gather_body(nb, nb_one, blk, nphase,
                 d2_hbm, rows_hbm, lanes_ref, out_ref,
                 d2_vmem, rows_sm, slot0, slot1, sems, d2_sem):
    j = pl.program_id(0)

    @pl.when(j == 0)
    def _prologue():
        pltpu.make_async_copy(d2_hbm, d2_vmem.at[:, 0, :], d2_sem).start()
        pltpu.make_async_copy(rows_hbm.at[pl.ds(0, blk)],
                              rows_sm.at[pl.ds(0, blk)], sems.at[0]).start()
        pltpu.make_async_copy(d2_hbm, d2_vmem.at[:, 0, :], d2_sem).wait()

    @pl.when(j + 1 < nb)
    def _prefetch():
        slot = (j + 1) % 2
        pltpu.make_async_copy(rows_hbm.at[pl.ds((j + 1) * blk, blk)],
                              rows_sm.at[pl.ds(slot * blk, blk)],
                              sems.at[slot]).start()

    slot = j % 2
    pltpu.make_async_copy(rows_hbm.at[pl.ds(j * blk, blk)],
                          rows_sm.at[pl.ds(slot * blk, blk)],
                          sems.at[slot]).wait()

    base = slot * blk
    iota2 = jax.lax.broadcasted_iota(jnp.int32, (128, 128), 1)

    def group(ci, acc):
        k0 = base + ci * (nphase * 128)
        lt = jax.lax.bitwise_and(jnp.transpose(
            lanes_ref[pl.ds(pl.multiple_of(nphase * ci, nphase), nphase), :],
            (1, 0)), 127)

        def stores(s):
            sl = slot0 if s % 2 == 0 else slot1
            for u in range(128):
                sl[pl.ds(u, 1)] = d2_vmem[rows_sm[k0 + s * 128 + u]]

        def maskadd(s, a):
            sl = slot0 if s % 2 == 0 else slot1
            return a + jnp.where(iota2 == lt[:, s:s + 1], sl[...], 0.0)

        stores(0)
        for s in range(nphase - 1):
            stores(s + 1)
            acc = maskadd(s, acc)
        acc = maskadd(nphase - 1, acc)
        return acc

    acc = jax.lax.fori_loop(
        0, blk // (nphase * 128), group, jnp.zeros((128, 128), jnp.float32))
    total = jnp.sum(acc, axis=0, keepdims=True)

    wt = jnp.where(j < nb_one, 1.0 - _ALPHA, _ALPHA).astype(jnp.float32)

    @pl.when(j == 0)
    def _init():
        out_ref[...] = jnp.zeros_like(out_ref)

    out_ref[...] += wt * total


def kernel(one_rows, one_cols, zero_rows, zero_cols, target, inp):
    m, n = inp.shape
    n_one = one_rows.shape[0]
    n_zero = zero_rows.shape[0]
    total = n_one + n_zero
    assert (m * n) % 128 == 0
    nrows = (m * n) // 128

    bm = m
    for cand in (256, 128, 64, 32, 16, 8):
        if m % cand == 0:
            bm = cand
            break
    d2 = pl.pallas_call(
        _d2_body,
        out_shape=jax.ShapeDtypeStruct((m, n), jnp.float32),
        grid=(m // bm,),
        in_specs=[pl.BlockSpec((bm, n), lambda i: (i, 0)),
                  pl.BlockSpec((bm, n), lambda i: (i, 0))],
        out_specs=pl.BlockSpec((bm, n), lambda i: (i, 0)),
        compiler_params=pltpu.CompilerParams(
            dimension_semantics=("arbitrary",)),
    )(inp.astype(jnp.float32), target.astype(jnp.float32))
    d2v = d2.reshape(nrows, 128)

    all_rows = jnp.concatenate([one_rows, zero_rows]).astype(jnp.int32)
    all_cols = jnp.concatenate([one_cols, zero_cols]).astype(jnp.int32)
    rows = all_rows * (n // 128) + jax.lax.shift_right_logical(all_cols, 7)
    lanes = all_cols.reshape(total // 128, 128)

    blk = 65536
    while blk > 1024 and (n_one % blk or n_zero % blk):
        blk //= 2
    assert blk % 1024 == 0 and n_one % blk == 0 and n_zero % blk == 0
    nb = total // blk
    nb_one = n_one // blk
    nphase = min(64, blk // 128)

    partials = pl.pallas_call(
        functools.partial(_gather_body, nb, nb_one, blk, nphase),
        out_shape=jax.ShapeDtypeStruct((1, 128), jnp.float32),
        grid=(nb,),
        in_specs=[pl.BlockSpec(memory_space=pl.ANY),
                  pl.BlockSpec(memory_space=pl.ANY),
                  pl.BlockSpec((blk // 128, 128), lambda j: (j, 0))],
        out_specs=pl.BlockSpec((1, 128), lambda j: (0, 0)),
        scratch_shapes=[
            pltpu.VMEM((nrows, 1, 128), jnp.float32),
            pltpu.SMEM((2 * blk,), jnp.int32),
            pltpu.VMEM((128, 128), jnp.float32),
            pltpu.VMEM((128, 128), jnp.float32),
            pltpu.SemaphoreType.DMA((2,)),
            pltpu.SemaphoreType.DMA,
        ],
        compiler_params=pltpu.CompilerParams(
            dimension_semantics=("arbitrary",)),
    )(d2v, rows, lanes)

    return jnp.sum(partials)


# speedup vs baseline: 1.0029x; 1.0029x over previous
import functools

import jax
import jax.numpy as jnp
from jax.experimental import pallas as pl
from jax.experimental.pallas import tpu as pltpu

_ALPHA = 0.2


def _d2_body(inp_ref, tgt_ref, out_ref):
    d = inp_ref[...] - tgt_ref[...]
    out_ref[...] = d * d


def _gather_body(nb, nb_one, blk, nphase,
                 d2_hbm, rows_hbm, lanes_ref, out_ref,
                 d2_vmem, rows_sm, slot0, slot1, sems, d2_sem):
    j = pl.program_id(0)

    @pl.when(j == 0)
    def _prologue():
        pltpu.make_async_copy(d2_hbm, d2_vmem.at[:, 0, :], d2_sem).start()
        pltpu.make_async_copy(rows_hbm.at[pl.ds(0, blk)],
                              rows_sm.at[pl.ds(0, blk)], sems.at[0]).start()
        pltpu.make_async_copy(d2_hbm, d2_vmem.at[:, 0, :], d2_sem).wait()

    @pl.when(j + 1 < nb)
    def _prefetch():
        slot = (j + 1) % 2
        pltpu.make_async_copy(rows_hbm.at[pl.ds((j + 1) * blk, blk)],
                              rows_sm.at[pl.ds(slot * blk, blk)],
                              sems.at[slot]).start()

    slot = j % 2
    pltpu.make_async_copy(rows_hbm.at[pl.ds(j * blk, blk)],
                          rows_sm.at[pl.ds(slot * blk, blk)],
                          sems.at[slot]).wait()

    base = slot * blk
    iota2 = jax.lax.broadcasted_iota(jnp.int32, (128, 128), 1)

    def group(ci, acc):
        k0 = base + ci * (nphase * 128)
        lt = jax.lax.bitwise_and(jnp.transpose(
            lanes_ref[pl.ds(pl.multiple_of(nphase * ci, nphase), nphase), :],
            (1, 0)), 127)

        def stores(s):
            sl = slot0 if s % 2 == 0 else slot1
            for u in range(128):
                sl[pl.ds(u, 1)] = d2_vmem[rows_sm[k0 + s * 128 + u]]

        def maskadd(s, a):
            sl = slot0 if s % 2 == 0 else slot1
            return a + jnp.where(iota2 == lt[:, s:s + 1], sl[...], 0.0)

        stores(0)
        for s in range(nphase - 1):
            stores(s + 1)
            acc = maskadd(s, acc)
        acc = maskadd(nphase - 1, acc)
        return acc

    acc = jax.lax.fori_loop(
        0, blk // (nphase * 128), group, jnp.zeros((128, 128), jnp.float32))
    total = jnp.sum(acc, axis=0, keepdims=True)

    wt = jnp.where(j < nb_one, 1.0 - _ALPHA, _ALPHA).astype(jnp.float32)

    @pl.when(j == 0)
    def _init():
        out_ref[...] = jnp.zeros_like(out_ref)

    out_ref[...] += wt * total


def kernel(one_rows, one_cols, zero_rows, zero_cols, target, inp):
    m, n = inp.shape
    n_one = one_rows.shape[0]
    n_zero = zero_rows.shape[0]
    total = n_one + n_zero
    assert (m * n) % 128 == 0
    nrows = (m * n) // 128

    bm = m
    for cand in (256, 128, 64, 32, 16, 8):
        if m % cand == 0:
            bm = cand
            break
    d2 = pl.pallas_call(
        _d2_body,
        out_shape=jax.ShapeDtypeStruct((m, n), jnp.float32),
        grid=(m // bm,),
        in_specs=[pl.BlockSpec((bm, n), lambda i: (i, 0)),
                  pl.BlockSpec((bm, n), lambda i: (i, 0))],
        out_specs=pl.BlockSpec((bm, n), lambda i: (i, 0)),
        compiler_params=pltpu.CompilerParams(
            dimension_semantics=("arbitrary",)),
    )(inp.astype(jnp.float32), target.astype(jnp.float32))
    d2v = d2.reshape(nrows, 128)

    all_rows = jnp.concatenate([one_rows, zero_rows]).astype(jnp.int32)
    all_cols = jnp.concatenate([one_cols, zero_cols]).astype(jnp.int32)
    rows = all_rows * (n // 128) + jax.lax.shift_right_logical(all_cols, 7)
    lanes = all_cols.reshape(total // 128, 128)

    blk = 65536
    while blk > 1024 and (n_one % blk or n_zero % blk):
        blk //= 2
    assert blk % 1024 == 0 and n_one % blk == 0 and n_zero % blk == 0
    nb = total // blk
    nb_one = n_one // blk
    nphase = min(128, blk // 128)

    partials = pl.pallas_call(
        functools.partial(_gather_body, nb, nb_one, blk, nphase),
        out_shape=jax.ShapeDtypeStruct((1, 128), jnp.float32),
        grid=(nb,),
        in_specs=[pl.BlockSpec(memory_space=pl.ANY),
                  pl.BlockSpec(memory_space=pl.ANY),
                  pl.BlockSpec((blk // 128, 128), lambda j: (j, 0))],
        out_specs=pl.BlockSpec((1, 128), lambda j: (0, 0)),
        scratch_shapes=[
            pltpu.VMEM((nrows, 1, 128), jnp.float32),
            pltpu.SMEM((2 * blk,), jnp.int32),
            pltpu.VMEM((128, 128), jnp.float32),
            pltpu.VMEM((128, 128), jnp.float32),
            pltpu.SemaphoreType.DMA((2,)),
            pltpu.SemaphoreType.DMA,
        ],
        compiler_params=pltpu.CompilerParams(
            dimension_semantics=("arbitrary",)),
    )(d2v, rows, lanes)

    return jnp.sum(partials)
